# baseline (speedup 1.0000x reference)
import functools

import jax
import jax.numpy as jnp
from jax.experimental import pallas as pl
from jax.experimental.pallas import tpu as pltpu

_INV_TEMP = 10.0
_EPS = 1e-12

_BM = 512
_BN = 1024


def _logits_kernel(x_ref, g_ref, o_ref):
    x = x_ref[...]
    g = g_ref[...]
    xn = x / jnp.maximum(jnp.sqrt(jnp.sum(x * x, axis=1, keepdims=True)), _EPS)
    gn = g / jnp.maximum(jnp.sqrt(jnp.sum(g * g, axis=1, keepdims=True)), _EPS)
    acc = jax.lax.dot_general(
        xn.astype(jnp.bfloat16),
        gn.astype(jnp.bfloat16),
        (((1,), (1,)), ((), ())),
        preferred_element_type=jnp.float32,
    )
    o_ref[...] = acc * _INV_TEMP


@functools.partial(jax.jit, static_argnames=())
def kernel(x, group_features):
    m, k = x.shape
    n, _ = group_features.shape
    grid = (m // _BM, n // _BN)
    return pl.pallas_call(
        _logits_kernel,
        grid=grid,
        in_specs=[
            pl.BlockSpec((_BM, k), lambda i, j: (i, 0)),
            pl.BlockSpec((_BN, k), lambda i, j: (j, 0)),
        ],
        out_specs=pl.BlockSpec((_BM, _BN), lambda i, j: (i, j)),
        out_shape=jax.ShapeDtypeStruct((m, n), jnp.float32),
        compiler_params=pltpu.CompilerParams(
            dimension_semantics=("parallel", "parallel"),
        ),
    )(x, group_features)


# speedup vs baseline: 2.0566x; 2.0566x over previous
import functools

import jax
import jax.numpy as jnp
from jax.experimental import pallas as pl
from jax.experimental.pallas import tpu as pltpu

_INV_TEMP = 10.0
_EPS = 1e-12

_BN = 512


def _logits_kernel(x_ref, g_ref, o_ref):
    x = x_ref[...]
    g = g_ref[...]
    xn = x / jnp.maximum(jnp.sqrt(jnp.sum(x * x, axis=1, keepdims=True)), _EPS)
    gn = g / jnp.maximum(jnp.sqrt(jnp.sum(g * g, axis=1, keepdims=True)), _EPS)
    acc = jax.lax.dot_general(
        xn.astype(jnp.bfloat16),
        gn.astype(jnp.bfloat16),
        (((1,), (1,)), ((), ())),
        preferred_element_type=jnp.float32,
    )
    o_ref[...] = acc * _INV_TEMP


@functools.partial(jax.jit, static_argnames=())
def kernel(x, group_features):
    m, k = x.shape
    n, _ = group_features.shape
    grid = (n // _BN,)
    return pl.pallas_call(
        _logits_kernel,
        grid=grid,
        in_specs=[
            pl.BlockSpec((m, k), lambda j: (0, 0)),
            pl.BlockSpec((_BN, k), lambda j: (j, 0)),
        ],
        out_specs=pl.BlockSpec((m, _BN), lambda j: (0, j)),
        out_shape=jax.ShapeDtypeStruct((m, n), jnp.float32),
        compiler_params=pltpu.CompilerParams(
            dimension_semantics=("arbitrary",),
        ),
    )(x, group_features)
